# baseline (speedup 1.0000x reference)
import jax
import jax.numpy as jnp
import numpy as np
from jax.experimental import pallas as pl
from jax.experimental.pallas import tpu as pltpu

D_MODEL = 1024
N_HEADS = 16
DH = D_MODEL // N_HEADS
B = 4
L = 2048
U = 40
QCH = 256
NCH = L // QCH
NHP = N_HEADS // 2
NEG = -1e30

_CNT3 = None


def _cnt3():
    global _CNT3
    if _CNT3 is None:
        with jax.ensure_compile_time_eval():
            idx = np.asarray(jax.random.randint(jax.random.key(42), (L, U), 0, L))
        cnt = np.zeros((L, L), np.float32)
        np.add.at(cnt, (np.arange(L)[:, None], idx), 1.0)
        cntT = cnt.T
        _CNT3 = jnp.asarray(
            np.stack([cntT[:, c * QCH:(c + 1) * QCH] for c in range(NCH)]))
    return _CNT3


def _proj_body(x_ref, c_ref, wq_ref, wk_ref, wv_ref, bq_ref, bk_ref, bv_ref,
               q_ref, k_ref, v_ref):
    f32 = jnp.float32
    x = x_ref[0]
    c = c_ref[0]
    q_ref[0] = jax.lax.dot_general(x, wq_ref[...], (((1,), (1,)), ((), ())),
                                   preferred_element_type=f32) + bq_ref[0][None, :]
    k_ref[0] = jax.lax.dot_general(c, wk_ref[...], (((1,), (1,)), ((), ())),
                                   preferred_element_type=f32) + bk_ref[0][None, :]
    v_ref[0] = jax.lax.dot_general(c, wv_ref[...], (((1,), (1,)), ((), ())),
                                   preferred_element_type=f32) + bv_ref[0][None, :]


def _attn_body(q_ref, k_ref, v_ref, cnt_ref, wo_ref, bo_ref, out_ref,
               m_scr, ctx_scr):
    f32 = jnp.float32
    hp = pl.program_id(1)
    QKV = []
    for i in range(2):
        sl = slice(i * DH, (i + 1) * DH)
        QKV.append((q_ref[0][:, sl], k_ref[0][:, sl], v_ref[0][:, sl]))

    for i, (Qm, Km, Vm) in enumerate(QKV):
        for qi in range(NCH):
            Qc = Qm[qi * QCH:(qi + 1) * QCH, :]
            St = jax.lax.dot_general(Km, Qc, (((1,), (1,)), ((), ())),
                                     preferred_element_type=f32)
            cc = cnt_ref[qi]
            mx = jnp.max(jnp.where(cc > 0.0, St, NEG), axis=0)
            sm = jnp.sum(St * cc, axis=0) * (1.0 / L)
            m_scr[NCH * i + qi, :] = mx - sm

    sub2n = jax.lax.broadcasted_iota(jnp.int32, (2 * NCH, QCH), 0)
    tok2n = (sub2n % NCH) * QCH + jax.lax.broadcasted_iota(
        jnp.int32, (2 * NCH, QCH), 1)
    row_of = sub2n // NCH
    sub_u = jax.lax.broadcasted_iota(jnp.int32, (U, 128), 0)
    lane_u = jax.lax.broadcasted_iota(jnp.int32, (U, 128), 1)

    def step(u, carry):
        Mc, idxc = carry
        upd_mask = jnp.zeros_like(Mc, dtype=jnp.bool_)
        for i in range(2):
            slab = Mc[NCH * i:NCH * (i + 1)]
            tslab = tok2n[NCH * i:NCH * (i + 1)]
            m_i = jnp.max(slab, axis=(0, 1), keepdims=True)
            idx_i = jnp.min(jnp.where(slab == m_i, tslab, L),
                            axis=(0, 1), keepdims=True)
            upd_mask = upd_mask | ((row_of == i) & (tok2n == idx_i))
            idxc = idxc + ((sub_u == u) & (lane_u == i)).astype(jnp.int32) * idx_i
        return jnp.where(upd_mask, NEG, Mc), idxc

    _, idxc = jax.lax.fori_loop(
        0, U, step,
        (m_scr[...], jnp.zeros((U, 128), jnp.int32)))

    lane_t = jax.lax.broadcasted_iota(jnp.int32, (U, L), 1)
    ctxs = []
    for i, (Qm, Km, Vm) in enumerate(QKV):
        oh = (lane_t == idxc[:, i:i + 1]).astype(f32)
        Qr = jax.lax.dot_general(oh, Qm, (((1,), (0,)), ((), ())),
                                 preferred_element_type=f32)
        sc = jax.lax.dot_general(Qr, Km, (((1,), (1,)), ((), ())),
                                 preferred_element_type=f32) * (1.0 / np.sqrt(DH))
        sc = sc - jnp.max(sc, axis=1, keepdims=True)
        e = jnp.exp(sc)
        attn = e / jnp.sum(e, axis=1, keepdims=True)
        upd = jax.lax.dot_general(attn, Vm, (((1,), (0,)), ((), ())),
                                  preferred_element_type=f32)
        vmean = jnp.mean(Vm, axis=0, keepdims=True)
        ctxs.append(jnp.broadcast_to(vmean, (L, DH))
                    + jax.lax.dot_general(oh, upd - vmean,
                                          (((0,), (0,)), ((), ())),
                                          preferred_element_type=f32))
    ctx_scr[hp] = jnp.concatenate(ctxs, axis=1)

    @pl.when(hp == NHP - 1)
    def _project():
        ctx_full = jnp.concatenate([ctx_scr[j] for j in range(NHP)], axis=1)
        out_ref[0] = jax.lax.dot_general(
            ctx_full, wo_ref[...], (((1,), (1,)), ((), ())),
            preferred_element_type=f32) + bo_ref[0][None, :]


@jax.jit
def _run(x, context, Wq, bq, Wk, bk, Wv, bv, Wo, bo, cnt3):
    f32 = jnp.float32
    LB = 512
    b2 = lambda v: v.reshape(1, D_MODEL)
    q, k, v = pl.pallas_call(
        _proj_body,
        grid=(B, L // LB),
        in_specs=[
            pl.BlockSpec((1, LB, D_MODEL), lambda b, l: (b, l, 0)),
            pl.BlockSpec((1, LB, D_MODEL), lambda b, l: (b, l, 0)),
            pl.BlockSpec((D_MODEL, D_MODEL), lambda b, l: (0, 0)),
            pl.BlockSpec((D_MODEL, D_MODEL), lambda b, l: (0, 0)),
            pl.BlockSpec((D_MODEL, D_MODEL), lambda b, l: (0, 0)),
            pl.BlockSpec((1, D_MODEL), lambda b, l: (0, 0)),
            pl.BlockSpec((1, D_MODEL), lambda b, l: (0, 0)),
            pl.BlockSpec((1, D_MODEL), lambda b, l: (0, 0)),
        ],
        out_specs=[
            pl.BlockSpec((1, LB, D_MODEL), lambda b, l: (b, l, 0)),
            pl.BlockSpec((1, LB, D_MODEL), lambda b, l: (b, l, 0)),
            pl.BlockSpec((1, LB, D_MODEL), lambda b, l: (b, l, 0)),
        ],
        out_shape=[jax.ShapeDtypeStruct((B, L, D_MODEL), f32)] * 3,
    )(x, context, Wq, Wk, Wv, b2(bq), b2(bk), b2(bv))

    out = pl.pallas_call(
        _attn_body,
        grid=(B, NHP),
        in_specs=[
            pl.BlockSpec((1, L, 2 * DH), lambda b, hp: (b, 0, hp)),
            pl.BlockSpec((1, L, 2 * DH), lambda b, hp: (b, 0, hp)),
            pl.BlockSpec((1, L, 2 * DH), lambda b, hp: (b, 0, hp)),
            pl.BlockSpec((NCH, L, QCH), lambda b, hp: (0, 0, 0)),
            pl.BlockSpec((D_MODEL, D_MODEL), lambda b, hp: (0, 0)),
            pl.BlockSpec((1, D_MODEL), lambda b, hp: (0, 0)),
        ],
        out_specs=pl.BlockSpec((1, L, D_MODEL), lambda b, hp: (b, 0, 0)),
        out_shape=jax.ShapeDtypeStruct((B, L, D_MODEL), f32),
        scratch_shapes=[pltpu.VMEM((2 * NCH, QCH), f32),
                        pltpu.VMEM((NHP, L, 2 * DH), f32)],
    )(q, k, v, cnt3, Wo, b2(bo))
    return out


def kernel(x, context, Wq, bq, Wk, bk, Wv, bv, Wo, bo):
    return _run(x, context, Wq, bq, Wk, bk, Wv, bv, Wo, bo, _cnt3())


# speedup vs baseline: 1.1038x; 1.1038x over previous
import jax
import jax.numpy as jnp
import numpy as np
from jax.experimental import pallas as pl
from jax.experimental.pallas import tpu as pltpu

D_MODEL = 1024
N_HEADS = 16
DH = D_MODEL // N_HEADS
B = 4
L = 2048
U = 40
QCH = 256
NCH = L // QCH
HPS = 8
NEG = -1e30

_CNT3 = None


def _cnt3():
    global _CNT3
    if _CNT3 is None:
        with jax.ensure_compile_time_eval():
            idx = np.asarray(jax.random.randint(jax.random.key(42), (L, U), 0, L))
        cnt = np.zeros((L, L), np.uint8)
        np.add.at(cnt, (np.arange(L)[:, None], idx), 1)
        cntT = cnt.T
        _CNT3 = jnp.asarray(
            np.stack([cntT[:, c * QCH:(c + 1) * QCH] for c in range(NCH)]))
    return _CNT3


def _proj_body(x_ref, c_ref, wq_ref, wk_ref, wv_ref, bq_ref, bk_ref, bv_ref,
               q_ref, k_ref, v_ref):
    f32 = jnp.float32
    x = x_ref[0]
    c = c_ref[0]
    q_ref[0] = jax.lax.dot_general(x, wq_ref[...], (((1,), (1,)), ((), ())),
                                   preferred_element_type=f32) + bq_ref[0][None, :]
    k_ref[0] = jax.lax.dot_general(c, wk_ref[...], (((1,), (1,)), ((), ())),
                                   preferred_element_type=f32) + bk_ref[0][None, :]
    v_ref[0] = jax.lax.dot_general(c, wv_ref[...], (((1,), (1,)), ((), ())),
                                   preferred_element_type=f32) + bv_ref[0][None, :]


def _attn_body(q_ref, k_ref, v_ref, cnt_ref, ctx_ref):
    f32 = jnp.float32
    QKV = [(q_ref[0][:, i * DH:(i + 1) * DH],
            k_ref[0][:, i * DH:(i + 1) * DH],
            v_ref[0][:, i * DH:(i + 1) * DH]) for i in range(HPS)]

    rows = [[] for _ in range(HPS)]
    for qi in range(NCH):
        cc = cnt_ref[qi].astype(f32)
        madd = jnp.where(cc > 0.0, 0.0, NEG)
        for i, (Qm, Km, _) in enumerate(QKV):
            Qc = Qm[qi * QCH:(qi + 1) * QCH, :]
            St = jax.lax.dot_general(Km, Qc, (((1,), (1,)), ((), ())),
                                     preferred_element_type=f32)
            mx = jnp.max(St + madd, axis=0)
            sm = jnp.sum(St * cc, axis=0) * (1.0 / L)
            rows[i].append((mx - sm)[None, :])
    Ms = [jnp.concatenate(r, axis=0) for r in rows]

    tok8 = (jax.lax.broadcasted_iota(jnp.int32, (NCH, QCH), 0) * QCH
            + jax.lax.broadcasted_iota(jnp.int32, (NCH, QCH), 1))
    sub_u = jax.lax.broadcasted_iota(jnp.int32, (U, 128), 0)
    lane_u = jax.lax.broadcasted_iota(jnp.int32, (U, 128), 1)

    def step(u, carry):
        slabs, idxc = carry
        new_slabs = []
        for i, slab in enumerate(slabs):
            m_i = jnp.max(slab, axis=(0, 1), keepdims=True)
            idx_i = jnp.min(jnp.where(slab == m_i, tok8, L),
                            axis=(0, 1), keepdims=True)
            new_slabs.append(jnp.where(tok8 == idx_i, NEG, slab))
            idxc = idxc + ((sub_u == u) & (lane_u == i)).astype(jnp.int32) * idx_i
        return tuple(new_slabs), idxc

    _, idxc = jax.lax.fori_loop(
        0, U, step, (tuple(Ms), jnp.zeros((U, 128), jnp.int32)))

    lane_t = jax.lax.broadcasted_iota(jnp.int32, (U, L), 1)
    for i, (Qm, Km, Vm) in enumerate(QKV):
        oh = (lane_t == idxc[:, i:i + 1]).astype(f32)
        Qr = jax.lax.dot_general(oh, Qm, (((1,), (0,)), ((), ())),
                                 preferred_element_type=f32)
        sc = jax.lax.dot_general(Qr, Km, (((1,), (1,)), ((), ())),
                                 preferred_element_type=f32) * (1.0 / np.sqrt(DH))
        sc = sc - jnp.max(sc, axis=1, keepdims=True)
        e = jnp.exp(sc)
        attn = e / jnp.sum(e, axis=1, keepdims=True)
        upd = jax.lax.dot_general(attn, Vm, (((1,), (0,)), ((), ())),
                                  preferred_element_type=f32)
        vmean = jnp.mean(Vm, axis=0, keepdims=True)
        ctx_ref[0, :, i * DH:(i + 1) * DH] = (
            jnp.broadcast_to(vmean, (L, DH))
            + jax.lax.dot_general(oh, upd - vmean, (((0,), (0,)), ((), ())),
                                  preferred_element_type=f32))


def _out_body(ctx_ref, wo_ref, bo_ref, out_ref):
    out_ref[0] = jax.lax.dot_general(
        ctx_ref[0], wo_ref[...], (((1,), (1,)), ((), ())),
        preferred_element_type=jnp.float32) + bo_ref[0][None, :]


@jax.jit
def _run(x, context, Wq, bq, Wk, bk, Wv, bv, Wo, bo, cnt3):
    f32 = jnp.float32
    LB = 512
    b2 = lambda v: v.reshape(1, D_MODEL)
    q, k, v = pl.pallas_call(
        _proj_body,
        grid=(B, L // LB),
        in_specs=[
            pl.BlockSpec((1, LB, D_MODEL), lambda b, l: (b, l, 0)),
            pl.BlockSpec((1, LB, D_MODEL), lambda b, l: (b, l, 0)),
            pl.BlockSpec((D_MODEL, D_MODEL), lambda b, l: (0, 0)),
            pl.BlockSpec((D_MODEL, D_MODEL), lambda b, l: (0, 0)),
            pl.BlockSpec((D_MODEL, D_MODEL), lambda b, l: (0, 0)),
            pl.BlockSpec((1, D_MODEL), lambda b, l: (0, 0)),
            pl.BlockSpec((1, D_MODEL), lambda b, l: (0, 0)),
            pl.BlockSpec((1, D_MODEL), lambda b, l: (0, 0)),
        ],
        out_specs=[
            pl.BlockSpec((1, LB, D_MODEL), lambda b, l: (b, l, 0)),
            pl.BlockSpec((1, LB, D_MODEL), lambda b, l: (b, l, 0)),
            pl.BlockSpec((1, LB, D_MODEL), lambda b, l: (b, l, 0)),
        ],
        out_shape=[jax.ShapeDtypeStruct((B, L, D_MODEL), f32)] * 3,
    )(x, context, Wq, Wk, Wv, b2(bq), b2(bk), b2(bv))

    HD = HPS * DH
    ctx = pl.pallas_call(
        _attn_body,
        grid=(B, N_HEADS // HPS),
        in_specs=[
            pl.BlockSpec((1, L, HD), lambda b, g: (b, 0, g)),
            pl.BlockSpec((1, L, HD), lambda b, g: (b, 0, g)),
            pl.BlockSpec((1, L, HD), lambda b, g: (b, 0, g)),
            pl.BlockSpec((NCH, L, QCH), lambda b, g: (0, 0, 0)),
        ],
        out_specs=pl.BlockSpec((1, L, HD), lambda b, g: (b, 0, g)),
        out_shape=jax.ShapeDtypeStruct((B, L, D_MODEL), f32),
    )(q, k, v, cnt3)

    out = pl.pallas_call(
        _out_body,
        grid=(B, L // LB),
        in_specs=[
            pl.BlockSpec((1, LB, D_MODEL), lambda b, l: (b, l, 0)),
            pl.BlockSpec((D_MODEL, D_MODEL), lambda b, l: (0, 0)),
            pl.BlockSpec((1, D_MODEL), lambda b, l: (0, 0)),
        ],
        out_specs=pl.BlockSpec((1, LB, D_MODEL), lambda b, l: (b, l, 0)),
        out_shape=jax.ShapeDtypeStruct((B, L, D_MODEL), f32),
    )(ctx, Wo, b2(bo))
    return out


def kernel(x, context, Wq, bq, Wk, bk, Wv, bv, Wo, bo):
    return _run(x, context, Wq, bq, Wk, bk, Wv, bv, Wo, bo, _cnt3())


# speedup vs baseline: 1.2836x; 1.1629x over previous
import jax
import jax.numpy as jnp
import numpy as np
from jax.experimental import pallas as pl
from jax.experimental.pallas import tpu as pltpu

D_MODEL = 1024
N_HEADS = 16
DH = D_MODEL // N_HEADS
B = 4
L = 2048
U = 40
QCH = 256
NCH = L // QCH
HPS = 8
NEG = -1e30

_CNT3 = None


def _cnt3():
    global _CNT3
    if _CNT3 is None:
        with jax.ensure_compile_time_eval():
            idx = np.asarray(jax.random.randint(jax.random.key(42), (L, U), 0, L))
        cnt = np.zeros((L, L), np.uint8)
        np.add.at(cnt, (np.arange(L)[:, None], idx), 1)
        cntT = cnt.T
        _CNT3 = jnp.asarray(
            np.stack([cntT[:, c * QCH:(c + 1) * QCH] for c in range(NCH)]))
    return _CNT3


def _proj_body(x_ref, c_ref, wq_ref, wk_ref, wv_ref, bq_ref, bk_ref, bv_ref,
               q_ref, k_ref, v_ref):
    f32 = jnp.float32
    x = x_ref[0]
    c = c_ref[0]
    q_ref[0] = jax.lax.dot_general(x, wq_ref[...], (((1,), (1,)), ((), ())),
                                   preferred_element_type=f32) + bq_ref[0][None, :]
    k_ref[0] = jax.lax.dot_general(c, wk_ref[...], (((1,), (1,)), ((), ())),
                                   preferred_element_type=f32) + bk_ref[0][None, :]
    v_ref[0] = jax.lax.dot_general(c, wv_ref[...], (((1,), (1,)), ((), ())),
                                   preferred_element_type=f32) + bv_ref[0][None, :]


def _attn_body(q_ref, k_ref, v_ref, cnt_ref, ctx_ref):
    f32 = jnp.float32
    QKV = [(q_ref[0][:, i * DH:(i + 1) * DH],
            k_ref[0][:, i * DH:(i + 1) * DH],
            v_ref[0][:, i * DH:(i + 1) * DH]) for i in range(HPS)]

    rows = [[] for _ in range(HPS)]
    for qi in range(NCH):
        cc = cnt_ref[qi].astype(f32)
        madd = jnp.where(cc > 0.0, 0.0, NEG)
        for i, (Qm, Km, _) in enumerate(QKV):
            Qc = Qm[qi * QCH:(qi + 1) * QCH, :]
            St = jax.lax.dot_general(Km, Qc, (((1,), (1,)), ((), ())),
                                     preferred_element_type=f32)
            mx = jnp.max(St + madd, axis=0)
            sm = jnp.sum(St * cc, axis=0) * (1.0 / L)
            rows[i].append((mx - sm)[None, :])
    M_all = jnp.concatenate([r for rs in rows for r in rs], axis=0)
    Mt0 = jnp.transpose(M_all)

    GW = HPS * NCH
    lane64 = jax.lax.broadcasted_iota(jnp.int32, (1, GW), 1)
    tokT = ((jax.lax.broadcasted_iota(jnp.int32, (QCH, GW), 1) % NCH) * QCH
            + jax.lax.broadcasted_iota(jnp.int32, (QCH, GW), 0))
    sub_u64 = jax.lax.broadcasted_iota(jnp.int32, (U, GW), 0)

    def seg_allreduce(x, op):
        for k in (1, 2, 4):
            partner = jnp.where((lane64 & k) == 0,
                                pltpu.roll(x, GW - k, 1), pltpu.roll(x, k, 1))
            x = op(x, partner)
        return x

    def step(u, carry):
        Mt, idxc = carry
        m = seg_allreduce(jnp.max(Mt, axis=0, keepdims=True), jnp.maximum)
        cand = jnp.where(Mt == m, tokT, L)
        idx = seg_allreduce(jnp.min(cand, axis=0, keepdims=True), jnp.minimum)
        Mt = jnp.where(tokT == idx, NEG, Mt)
        idxc = idxc + (sub_u64 == u).astype(jnp.int32) * idx
        return Mt, idxc

    _, idxc = jax.lax.fori_loop(
        0, U, step, (Mt0, jnp.zeros((U, GW), jnp.int32)))

    lane_t = jax.lax.broadcasted_iota(jnp.int32, (U, L), 1)
    for i, (Qm, Km, Vm) in enumerate(QKV):
        oh = (lane_t == idxc[:, NCH * i:NCH * i + 1]).astype(f32)
        Qr = jax.lax.dot_general(oh, Qm, (((1,), (0,)), ((), ())),
                                 preferred_element_type=f32)
        sc = jax.lax.dot_general(Qr, Km, (((1,), (1,)), ((), ())),
                                 preferred_element_type=f32) * (1.0 / np.sqrt(DH))
        sc = sc - jnp.max(sc, axis=1, keepdims=True)
        e = jnp.exp(sc)
        attn = e / jnp.sum(e, axis=1, keepdims=True)
        upd = jax.lax.dot_general(attn, Vm, (((1,), (0,)), ((), ())),
                                  preferred_element_type=f32)
        vmean = jnp.mean(Vm, axis=0, keepdims=True)
        ctx_ref[0, :, i * DH:(i + 1) * DH] = (
            jnp.broadcast_to(vmean, (L, DH))
            + jax.lax.dot_general(oh, upd - vmean, (((0,), (0,)), ((), ())),
                                  preferred_element_type=f32))


def _out_body(ctx_ref, wo_ref, bo_ref, out_ref):
    out_ref[0] = jax.lax.dot_general(
        ctx_ref[0], wo_ref[...], (((1,), (1,)), ((), ())),
        preferred_element_type=jnp.float32) + bo_ref[0][None, :]


@jax.jit
def _run(x, context, Wq, bq, Wk, bk, Wv, bv, Wo, bo, cnt3):
    f32 = jnp.float32
    LB = 512
    b2 = lambda v: v.reshape(1, D_MODEL)
    q, k, v = pl.pallas_call(
        _proj_body,
        grid=(B, L // LB),
        in_specs=[
            pl.BlockSpec((1, LB, D_MODEL), lambda b, l: (b, l, 0)),
            pl.BlockSpec((1, LB, D_MODEL), lambda b, l: (b, l, 0)),
            pl.BlockSpec((D_MODEL, D_MODEL), lambda b, l: (0, 0)),
            pl.BlockSpec((D_MODEL, D_MODEL), lambda b, l: (0, 0)),
            pl.BlockSpec((D_MODEL, D_MODEL), lambda b, l: (0, 0)),
            pl.BlockSpec((1, D_MODEL), lambda b, l: (0, 0)),
            pl.BlockSpec((1, D_MODEL), lambda b, l: (0, 0)),
            pl.BlockSpec((1, D_MODEL), lambda b, l: (0, 0)),
        ],
        out_specs=[
            pl.BlockSpec((1, LB, D_MODEL), lambda b, l: (b, l, 0)),
            pl.BlockSpec((1, LB, D_MODEL), lambda b, l: (b, l, 0)),
            pl.BlockSpec((1, LB, D_MODEL), lambda b, l: (b, l, 0)),
        ],
        out_shape=[jax.ShapeDtypeStruct((B, L, D_MODEL), f32)] * 3,
    )(x, context, Wq, Wk, Wv, b2(bq), b2(bk), b2(bv))

    HD = HPS * DH
    ctx = pl.pallas_call(
        _attn_body,
        grid=(B, N_HEADS // HPS),
        in_specs=[
            pl.BlockSpec((1, L, HD), lambda b, g: (b, 0, g)),
            pl.BlockSpec((1, L, HD), lambda b, g: (b, 0, g)),
            pl.BlockSpec((1, L, HD), lambda b, g: (b, 0, g)),
            pl.BlockSpec((NCH, L, QCH), lambda b, g: (0, 0, 0)),
        ],
        out_specs=pl.BlockSpec((1, L, HD), lambda b, g: (b, 0, g)),
        out_shape=jax.ShapeDtypeStruct((B, L, D_MODEL), f32),
    )(q, k, v, cnt3)

    out = pl.pallas_call(
        _out_body,
        grid=(B, L // LB),
        in_specs=[
            pl.BlockSpec((1, LB, D_MODEL), lambda b, l: (b, l, 0)),
            pl.BlockSpec((D_MODEL, D_MODEL), lambda b, l: (0, 0)),
            pl.BlockSpec((1, D_MODEL), lambda b, l: (0, 0)),
        ],
        out_specs=pl.BlockSpec((1, LB, D_MODEL), lambda b, l: (b, l, 0)),
        out_shape=jax.ShapeDtypeStruct((B, L, D_MODEL), f32),
    )(ctx, Wo, b2(bo))
    return out


def kernel(x, context, Wq, bq, Wk, bk, Wv, bv, Wo, bo):
    return _run(x, context, Wq, bq, Wk, bk, Wv, bv, Wo, bo, _cnt3())
